# baseline (speedup 1.0000x reference)
import jax
import jax.numpy as jnp
from jax import lax
from jax.experimental import pallas as pl
from jax.experimental.pallas import tpu as pltpu
from jax.experimental.pallas import tpu_sc as plsc

ENT_SIZE = 64
REL_SIZE = 64
MARGIN = 1.0
BATCH = 4096
NC, NS = 2, 16
NW = NC * NS
BPW = BATCH // NW
CHUNK = 8
NCHUNK = BPW // CHUNK
L = 16
EV = ENT_SIZE // L
OUTW = 128


def _rsqrt_nr(x):
    xh = x * 0.5
    i = lax.bitcast_convert_type(x, jnp.int32)
    i = jnp.int32(0x5F3759DF) - lax.shift_right_logical(i, 1)
    y = lax.bitcast_convert_type(i, jnp.float32)
    for _ in range(3):
        y = y * (1.5 - xh * y * y)
    return y


def _lanesum(x, ii):
    for sh in (8, 4, 2, 1):
        x = x + jnp.take(x, ii ^ sh)
    return x


def _l2n(vs, ii):
    ss = vs[0] * vs[0]
    for j in range(1, EV):
        ss = ss + vs[j] * vs[j]
    total = _lanesum(ss, ii)
    inv = _rsqrt_nr(jnp.maximum(total, 1e-12))
    return [v * inv for v in vs]


def _sc_body(tm_hbm, erow_hbm, rrow_hbm,
             pr_hbm,
             out_hbm,
             iv_pr,
             eb_h, eb_t, eb_nh, eb_nt, eb_r, eb_nr,
             mbuf, outv,
             msem0, msem1, esem0, esem1):
    wid = lax.axis_index("s") * NC + lax.axis_index("c")
    base = wid * BPW
    ii = lax.iota(jnp.int32, L)

    ebufs = (eb_h, eb_t, eb_nh, eb_nt, eb_r, eb_nr)
    srcs = ((erow_hbm, 0), (erow_hbm, BATCH), (erow_hbm, 2 * BATCH),
            (erow_hbm, 3 * BATCH), (rrow_hbm, 0), (rrow_hbm, BATCH))

    pltpu.sync_copy(pr_hbm.at[pl.ds(base, BPW)], iv_pr)

    def issue(c, half, msem, esem):
        rb = half * CHUNK
        pltpu.async_copy(tm_hbm.at[iv_pr.at[pl.ds(c * CHUNK, CHUNK)]],
                         mbuf.at[pl.ds(rb, CHUNK)], msem)
        for (src, off), ebuf in zip(srcs, ebufs):
            pltpu.async_copy(src.at[pl.ds(off + base + c * CHUNK, CHUNK)],
                             ebuf.at[pl.ds(rb, CHUNK)], esem)

    def drain(half, msem, esem):
        rb = half * CHUNK
        pltpu.make_async_copy(tm_hbm.at[pl.ds(0, CHUNK)],
                              mbuf.at[pl.ds(rb, CHUNK)], msem).wait()
        for (src, _), ebuf in zip(srcs, ebufs):
            pltpu.make_async_copy(src.at[pl.ds(0, CHUNK)],
                                  ebuf.at[pl.ds(rb, CHUNK)], esem).wait()

    def compute(rb, acc):
        def sample(s, acc):
            r = rb + s
            ph = [jnp.zeros((L,), jnp.float32) for _ in range(EV)]
            pt = [jnp.zeros((L,), jnp.float32) for _ in range(EV)]
            nh = [jnp.zeros((L,), jnp.float32) for _ in range(EV)]
            nt = [jnp.zeros((L,), jnp.float32) for _ in range(EV)]
            for eo in range(EV):
                hv = eb_h[r, pl.ds(eo * L, L)]
                tv = eb_t[r, pl.ds(eo * L, L)]
                nhv = eb_nh[r, pl.ds(eo * L, L)]
                ntv = eb_nt[r, pl.ds(eo * L, L)]
                off0 = eo * (L * REL_SIZE)
                for el in range(L):
                    m = [mbuf[r, pl.ds(off0 + el * REL_SIZE + j * L, L)]
                         for j in range(EV)]
                    hb = jnp.full((L,), hv[el])
                    tb = jnp.full((L,), tv[el])
                    nhb = jnp.full((L,), nhv[el])
                    ntb = jnp.full((L,), ntv[el])
                    for j in range(EV):
                        ph[j] = ph[j] + hb * m[j]
                        pt[j] = pt[j] + tb * m[j]
                        nh[j] = nh[j] + nhb * m[j]
                        nt[j] = nt[j] + ntb * m[j]

            phn = _l2n(ph, ii)
            ptn = _l2n(pt, ii)
            nhn = _l2n(nh, ii)
            ntn = _l2n(nt, ii)
            prn = _l2n([eb_r[r, pl.ds(j * L, L)] for j in range(EV)], ii)
            nrn = _l2n([eb_nr[r, pl.ds(j * L, L)] for j in range(EV)], ii)

            pd = jnp.abs(phn[0] + prn[0] - ptn[0])
            nd = jnp.abs(nhn[0] + nrn[0] - ntn[0])
            for j in range(1, EV):
                pd = pd + jnp.abs(phn[j] + prn[j] - ptn[j])
                nd = nd + jnp.abs(nhn[j] + nrn[j] - ntn[j])
            p_sc = _lanesum(pd, ii)
            n_sc = _lanesum(nd, ii)
            return acc + jnp.maximum(p_sc - n_sc + MARGIN, 0.0)

        def pair(s2, acc):
            return sample(2 * s2 + 1, sample(2 * s2, acc))

        return lax.fori_loop(0, CHUNK // 2, pair, acc)

    issue(0, 0, msem0, esem0)

    def body(c, acc):
        par = c & 1

        @pl.when(par == 0)
        def _():
            @pl.when(c < NCHUNK - 1)
            def _():
                issue(c + 1, 1, msem1, esem1)
            drain(0, msem0, esem0)

        @pl.when(par == 1)
        def _():
            @pl.when(c < NCHUNK - 1)
            def _():
                issue(c + 1, 0, msem0, esem0)
            drain(1, msem1, esem1)

        return compute(par * CHUNK, acc)

    acc = lax.fori_loop(0, NCHUNK, body, jnp.zeros((L,), jnp.float32))
    outv[pl.ds(0, L)] = acc
    pltpu.sync_copy(outv, out_hbm.at[wid])


def kernel(ent_embeddings, rel_embeddings, transfer_matrix,
           pos_h, pos_t, pos_r, neg_h, neg_t, neg_r):
    pos_h, pos_t, pos_r, neg_h, neg_t, neg_r = (
        x.astype(jnp.int32) for x in (pos_h, pos_t, pos_r,
                                      neg_h, neg_t, neg_r))
    eidx = jnp.concatenate([pos_h, pos_t, neg_h, neg_t])
    ridx = jnp.concatenate([pos_r, neg_r])
    erows = ent_embeddings.at[eidx].get(mode="promise_in_bounds")
    rrows = rel_embeddings.at[ridx].get(mode="promise_in_bounds")

    mesh = plsc.VectorSubcoreMesh(core_axis_name="c", subcore_axis_name="s")
    ebuf_t = pltpu.VMEM((2 * CHUNK, ENT_SIZE), jnp.float32)
    run = pl.kernel(
        _sc_body,
        out_type=jax.ShapeDtypeStruct((NW, OUTW), jnp.float32),
        mesh=mesh,
        scratch_types=(
            [pltpu.VMEM((BPW,), jnp.int32)]
            + [ebuf_t] * 6
            + [pltpu.VMEM((2 * CHUNK, ENT_SIZE * REL_SIZE), jnp.float32)]
            + [pltpu.VMEM((OUTW,), jnp.float32)]
            + [pltpu.SemaphoreType.DMA] * 4
        ),
    )
    partials = run(transfer_matrix, erows, rrows, pos_r)
    return jnp.sum(partials[:, 0]) / BATCH


# speedup vs baseline: 1.0293x; 1.0293x over previous
import jax
import jax.numpy as jnp
from jax import lax
from jax.experimental import pallas as pl
from jax.experimental.pallas import tpu as pltpu
from jax.experimental.pallas import tpu_sc as plsc

ENT_SIZE = 64
REL_SIZE = 64
MARGIN = 1.0
BATCH = 4096
NC, NS = 2, 16
NW = NC * NS
BPW = BATCH // NW
CHUNK = 8
NCHUNK = BPW // CHUNK
L = 16
EV = ENT_SIZE // L
OUTW = 128


def _rsqrt_nr(x):
    xh = x * 0.5
    i = lax.bitcast_convert_type(x, jnp.int32)
    i = jnp.int32(0x5F3759DF) - lax.shift_right_logical(i, 1)
    y = lax.bitcast_convert_type(i, jnp.float32)
    for _ in range(3):
        y = y * (1.5 - xh * y * y)
    return y


def _lanesum(x, ii):
    for sh in (8, 4, 2, 1):
        x = x + jnp.take(x, ii ^ sh)
    return x


def _l2n(vs, ii):
    ss = vs[0] * vs[0]
    for j in range(1, EV):
        ss = ss + vs[j] * vs[j]
    total = _lanesum(ss, ii)
    inv = _rsqrt_nr(jnp.maximum(total, 1e-12))
    return [v * inv for v in vs]


def _sc_body(tm_hbm, erow_hbm, rrow_hbm,
             pr_hbm,
             out_hbm,
             iv_pr,
             eb_h, eb_t, eb_nh, eb_nt, eb_r, eb_nr,
             mbuf, outv,
             msem0, msem1, esem0, esem1):
    wid = lax.axis_index("s") * NC + lax.axis_index("c")
    base = wid * BPW
    ii = lax.iota(jnp.int32, L)

    ebufs = (eb_h, eb_t, eb_nh, eb_nt, eb_r, eb_nr)
    srcs = ((erow_hbm, 0), (erow_hbm, BATCH), (erow_hbm, 2 * BATCH),
            (erow_hbm, 3 * BATCH), (rrow_hbm, 0), (rrow_hbm, BATCH))

    pltpu.sync_copy(pr_hbm.at[pl.ds(base, BPW)], iv_pr)

    def issue(c, half, msem, esem):
        rb = half * CHUNK
        pltpu.async_copy(tm_hbm.at[iv_pr.at[pl.ds(c * CHUNK, CHUNK)]],
                         mbuf.at[pl.ds(rb, CHUNK)], msem)
        for (src, off), ebuf in zip(srcs, ebufs):
            pltpu.async_copy(src.at[pl.ds(off + base + c * CHUNK, CHUNK)],
                             ebuf.at[pl.ds(rb, CHUNK)], esem)

    def drain(half, msem, esem):
        rb = half * CHUNK
        pltpu.make_async_copy(tm_hbm.at[pl.ds(0, CHUNK)],
                              mbuf.at[pl.ds(rb, CHUNK)], msem).wait()
        for (src, _), ebuf in zip(srcs, ebufs):
            pltpu.make_async_copy(src.at[pl.ds(0, CHUNK)],
                                  ebuf.at[pl.ds(rb, CHUNK)], esem).wait()

    def compute(rb, acc):
        def sample(s, acc):
            r = rb + s
            zeros = tuple(jnp.zeros((L,), jnp.float32) for _ in range(EV))

            def ebody(eo, carry):
                ph, pt, nh, nt = (list(x) for x in carry)
                hv = eb_h[r, pl.ds(eo * L, L)]
                tv = eb_t[r, pl.ds(eo * L, L)]
                nhv = eb_nh[r, pl.ds(eo * L, L)]
                ntv = eb_nt[r, pl.ds(eo * L, L)]
                off0 = eo * (L * REL_SIZE)
                for el in range(L):
                    m = [mbuf[r, pl.ds(off0 + el * REL_SIZE + j * L, L)]
                         for j in range(EV)]
                    hb = jnp.full((L,), hv[el])
                    tb = jnp.full((L,), tv[el])
                    nhb = jnp.full((L,), nhv[el])
                    ntb = jnp.full((L,), ntv[el])
                    for j in range(EV):
                        ph[j] = ph[j] + hb * m[j]
                        pt[j] = pt[j] + tb * m[j]
                        nh[j] = nh[j] + nhb * m[j]
                        nt[j] = nt[j] + ntb * m[j]
                return tuple(ph), tuple(pt), tuple(nh), tuple(nt)

            ph, pt, nh, nt = lax.fori_loop(
                0, EV, ebody, (zeros, zeros, zeros, zeros))
            ph, pt, nh, nt = list(ph), list(pt), list(nh), list(nt)

            phn = _l2n(ph, ii)
            ptn = _l2n(pt, ii)
            nhn = _l2n(nh, ii)
            ntn = _l2n(nt, ii)
            prn = _l2n([eb_r[r, pl.ds(j * L, L)] for j in range(EV)], ii)
            nrn = _l2n([eb_nr[r, pl.ds(j * L, L)] for j in range(EV)], ii)

            pd = jnp.abs(phn[0] + prn[0] - ptn[0])
            nd = jnp.abs(nhn[0] + nrn[0] - ntn[0])
            for j in range(1, EV):
                pd = pd + jnp.abs(phn[j] + prn[j] - ptn[j])
                nd = nd + jnp.abs(nhn[j] + nrn[j] - ntn[j])
            p_sc = _lanesum(pd, ii)
            n_sc = _lanesum(nd, ii)
            return acc + jnp.maximum(p_sc - n_sc + MARGIN, 0.0)

        return lax.fori_loop(0, CHUNK, sample, acc)

    issue(0, 0, msem0, esem0)

    def body(c, acc):
        par = c & 1

        @pl.when(par == 0)
        def _():
            @pl.when(c < NCHUNK - 1)
            def _():
                issue(c + 1, 1, msem1, esem1)
            drain(0, msem0, esem0)

        @pl.when(par == 1)
        def _():
            @pl.when(c < NCHUNK - 1)
            def _():
                issue(c + 1, 0, msem0, esem0)
            drain(1, msem1, esem1)

        return compute(par * CHUNK, acc)

    acc = lax.fori_loop(0, NCHUNK, body, jnp.zeros((L,), jnp.float32))
    outv[pl.ds(0, L)] = acc
    pltpu.sync_copy(outv, out_hbm.at[wid])


def kernel(ent_embeddings, rel_embeddings, transfer_matrix,
           pos_h, pos_t, pos_r, neg_h, neg_t, neg_r):
    pos_h, pos_t, pos_r, neg_h, neg_t, neg_r = (
        x.astype(jnp.int32) for x in (pos_h, pos_t, pos_r,
                                      neg_h, neg_t, neg_r))
    eidx = jnp.concatenate([pos_h, pos_t, neg_h, neg_t])
    ridx = jnp.concatenate([pos_r, neg_r])
    erows = ent_embeddings.at[eidx].get(mode="promise_in_bounds")
    rrows = rel_embeddings.at[ridx].get(mode="promise_in_bounds")

    mesh = plsc.VectorSubcoreMesh(core_axis_name="c", subcore_axis_name="s")
    ebuf_t = pltpu.VMEM((2 * CHUNK, ENT_SIZE), jnp.float32)
    run = pl.kernel(
        _sc_body,
        out_type=jax.ShapeDtypeStruct((NW, OUTW), jnp.float32),
        mesh=mesh,
        scratch_types=(
            [pltpu.VMEM((BPW,), jnp.int32)]
            + [ebuf_t] * 6
            + [pltpu.VMEM((2 * CHUNK, ENT_SIZE * REL_SIZE), jnp.float32)]
            + [pltpu.VMEM((OUTW,), jnp.float32)]
            + [pltpu.SemaphoreType.DMA] * 4
        ),
    )
    partials = run(transfer_matrix, erows, rrows, pos_r)
    return jnp.sum(partials[:, 0]) / BATCH


# speedup vs baseline: 1.3641x; 1.3253x over previous
import jax
import jax.numpy as jnp
from jax import lax
from jax.experimental import pallas as pl
from jax.experimental.pallas import tpu as pltpu
from jax.experimental.pallas import tpu_sc as plsc

ENT_SIZE = 64
REL_SIZE = 64
MARGIN = 1.0
BATCH = 4096
NC, NS = 2, 16
NW = NC * NS
BPW = BATCH // NW
CHUNK = 8
NCHUNK = BPW // CHUNK
L = 16
EV = ENT_SIZE // L
OUTW = 128


def _rsqrt_nr(x):
    xh = x * 0.5
    i = lax.bitcast_convert_type(x, jnp.int32)
    i = jnp.int32(0x5F3759DF) - lax.shift_right_logical(i, 1)
    y = lax.bitcast_convert_type(i, jnp.float32)
    for _ in range(3):
        y = y * (1.5 - xh * y * y)
    return y


def _lanesum(x, ii):
    for sh in (8, 4, 2, 1):
        x = x + jnp.take(x, ii ^ sh)
    return x


def _l2n(vs, ii):
    ss = vs[0] * vs[0]
    for j in range(1, EV):
        ss = ss + vs[j] * vs[j]
    total = _lanesum(ss, ii)
    inv = _rsqrt_nr(jnp.maximum(total, 1e-12))
    return [v * inv for v in vs]


def _sc_body(tm_hbm, erow_hbm, rrow_hbm,
             pr_hbm,
             out_hbm,
             iv_pr,
             eb_h, eb_t, eb_nh, eb_nt, eb_r, eb_nr,
             mbuf, outv,
             msem0, msem1, esem0, esem1):
    wid = lax.axis_index("s") * NC + lax.axis_index("c")
    base = wid * BPW
    ii = lax.iota(jnp.int32, L)

    ebufs = (eb_h, eb_t, eb_nh, eb_nt, eb_r, eb_nr)
    srcs = ((erow_hbm, 0), (erow_hbm, BATCH), (erow_hbm, 2 * BATCH),
            (erow_hbm, 3 * BATCH), (rrow_hbm, 0), (rrow_hbm, BATCH))

    pltpu.sync_copy(pr_hbm.at[pl.ds(base, BPW)], iv_pr)

    def issue(c, half, msem, esem):
        rb = half * CHUNK
        pltpu.async_copy(tm_hbm.at[iv_pr.at[pl.ds(c * CHUNK, CHUNK)]],
                         mbuf.at[pl.ds(rb, CHUNK)], msem)
        for (src, off), ebuf in zip(srcs, ebufs):
            pltpu.async_copy(src.at[pl.ds(off + base + c * CHUNK, CHUNK)],
                             ebuf.at[pl.ds(rb, CHUNK)], esem)

    def drain(half, msem, esem):
        rb = half * CHUNK
        pltpu.make_async_copy(tm_hbm.at[pl.ds(0, CHUNK)],
                              mbuf.at[pl.ds(rb, CHUNK)], msem).wait()
        for (src, _), ebuf in zip(srcs, ebufs):
            pltpu.make_async_copy(src.at[pl.ds(0, CHUNK)],
                                  ebuf.at[pl.ds(rb, CHUNK)], esem).wait()

    def compute(rb, acc):
        def sample(s, acc):
            r = rb + s
            ph = [jnp.zeros((L,), jnp.float32) for _ in range(EV)]
            pt = [jnp.zeros((L,), jnp.float32) for _ in range(EV)]
            nh = [jnp.zeros((L,), jnp.float32) for _ in range(EV)]
            nt = [jnp.zeros((L,), jnp.float32) for _ in range(EV)]
            for eo in range(EV):
                hv = eb_h[r, pl.ds(eo * L, L)]
                tv = eb_t[r, pl.ds(eo * L, L)]
                nhv = eb_nh[r, pl.ds(eo * L, L)]
                ntv = eb_nt[r, pl.ds(eo * L, L)]
                off0 = eo * (L * REL_SIZE)
                for el in range(L):
                    m = [mbuf[r, pl.ds(off0 + el * REL_SIZE + j * L, L)]
                         for j in range(EV)]
                    hb = jnp.full((L,), hv[el])
                    tb = jnp.full((L,), tv[el])
                    nhb = jnp.full((L,), nhv[el])
                    ntb = jnp.full((L,), ntv[el])
                    for j in range(EV):
                        ph[j] = ph[j] + hb * m[j]
                        pt[j] = pt[j] + tb * m[j]
                        nh[j] = nh[j] + nhb * m[j]
                        nt[j] = nt[j] + ntb * m[j]

            phn = _l2n(ph, ii)
            ptn = _l2n(pt, ii)
            nhn = _l2n(nh, ii)
            ntn = _l2n(nt, ii)
            prn = _l2n([eb_r[r, pl.ds(j * L, L)] for j in range(EV)], ii)
            nrn = _l2n([eb_nr[r, pl.ds(j * L, L)] for j in range(EV)], ii)

            pd = jnp.abs(phn[0] + prn[0] - ptn[0])
            nd = jnp.abs(nhn[0] + nrn[0] - ntn[0])
            for j in range(1, EV):
                pd = pd + jnp.abs(phn[j] + prn[j] - ptn[j])
                nd = nd + jnp.abs(nhn[j] + nrn[j] - ntn[j])
            p_sc = _lanesum(pd, ii)
            n_sc = _lanesum(nd, ii)
            return acc + jnp.maximum(p_sc - n_sc + MARGIN, 0.0)

        return lax.fori_loop(0, CHUNK, sample, acc)

    issue(0, 0, msem0, esem0)

    def body(c, acc):
        par = c & 1

        @pl.when(par == 0)
        def _():
            @pl.when(c < NCHUNK - 1)
            def _():
                issue(c + 1, 1, msem1, esem1)
            drain(0, msem0, esem0)

        @pl.when(par == 1)
        def _():
            @pl.when(c < NCHUNK - 1)
            def _():
                issue(c + 1, 0, msem0, esem0)
            drain(1, msem1, esem1)

        return compute(par * CHUNK, acc)

    acc = lax.fori_loop(0, NCHUNK, body, jnp.zeros((L,), jnp.float32))
    outv[pl.ds(0, L)] = acc
    pltpu.sync_copy(outv, out_hbm.at[wid])


def kernel(ent_embeddings, rel_embeddings, transfer_matrix,
           pos_h, pos_t, pos_r, neg_h, neg_t, neg_r):
    pos_h, pos_t, pos_r, neg_h, neg_t, neg_r = (
        x.astype(jnp.int32) for x in (pos_h, pos_t, pos_r,
                                      neg_h, neg_t, neg_r))
    eidx = jnp.concatenate([pos_h, pos_t, neg_h, neg_t])
    ridx = jnp.concatenate([pos_r, neg_r])
    erows = ent_embeddings.at[eidx].get(mode="promise_in_bounds")
    rrows = rel_embeddings.at[ridx].get(mode="promise_in_bounds")

    mesh = plsc.VectorSubcoreMesh(core_axis_name="c", subcore_axis_name="s")
    ebuf_t = pltpu.VMEM((2 * CHUNK, ENT_SIZE), jnp.float32)
    run = pl.kernel(
        _sc_body,
        out_type=jax.ShapeDtypeStruct((NW, OUTW), jnp.float32),
        mesh=mesh,
        scratch_types=(
            [pltpu.VMEM((BPW,), jnp.int32)]
            + [ebuf_t] * 6
            + [pltpu.VMEM((2 * CHUNK, ENT_SIZE * REL_SIZE), jnp.float32)]
            + [pltpu.VMEM((OUTW,), jnp.float32)]
            + [pltpu.SemaphoreType.DMA] * 4
        ),
    )
    partials = run(transfer_matrix, erows, rrows, pos_r)
    return jnp.sum(partials[:, 0]) / BATCH
